# Initial kernel scaffold; baseline (speedup 1.0000x reference)
import jax
import jax.numpy as jnp
from jax.experimental import pallas as pl


def kernel(vertices, faces):
    raise NotImplementedError("write your pallas kernel here")


# speedup vs baseline: 2.2574x; 2.2574x over previous
import jax
import jax.numpy as jnp
from jax import lax
from jax.experimental import pallas as pl
from jax.experimental.pallas import tpu as pltpu
from jax.experimental.pallas import tpu_sc as plsc

V = 10000
F = 20000
WEIGHT = 0.1
L = 16
NC, NS = 2, 16
NW = NC * NS
FPT = 640
FPAD = FPT * NW
GROUPS = FPT // L
CHUNK = 6 * L
VPAD = 10240
ACC = 4 * VPAD
SLICE = ACC // NS
SENT = V * V
TSIZE = SENT + 8

_PAIRS = ((0, 1), (0, 2), (1, 0), (1, 2), (2, 0), (2, 1))


def _expand_group(fwin, fbase, g, iot):
    lf = g * L + iot
    widx = 3 * lf
    fv = [plsc.load_gather(fwin, [widx + d]) for d in range(3)]
    fid = fbase + lf
    valid = fid < F
    rc = [(fv[a], fv[b]) for (a, b) in _PAIRS]
    return fid, valid, rc


def _stage1_body(faces_hbm, t_hbm, fwin, ka, ea, kb, eb, sem):
    cid = lax.axis_index("c")
    sid = lax.axis_index("s")
    wid = sid * NC + cid
    fbase = wid * FPT
    pltpu.sync_copy(faces_hbm.at[pl.ds(fbase * 3, FPT * 3)], fwin)
    iot = lax.iota(jnp.int32, L)

    def fill(g, krow, erow):
        fid, valid, rc = _expand_group(fwin, fbase, g, iot)
        for p, (r, c) in enumerate(rc):
            krow[pl.ds(p * L, L)] = jnp.where(valid, r * V + c, SENT)
            erow[pl.ds(p * L, L)] = fid * 6 + p

    def body(i, carry):
        fill(2 * i, ka, ea)
        da = pltpu.async_copy(ea, t_hbm.at[ka], sem)
        fill(2 * i + 1, kb, eb)
        db = pltpu.async_copy(eb, t_hbm.at[kb], sem)
        da.wait()
        db.wait()
        return carry

    lax.fori_loop(0, GROUPS // 2, body, 0)


def _stage2_body(faces_hbm, vpl_hbm, t_hbm, part_hbm, fwin, vbuf, acc,
                 ka, ra, ca, ta, kb, rb, cb, tb, spm, tmpb, sumb, sem):
    cid = lax.axis_index("c")
    sid = lax.axis_index("s")
    wid = sid * NC + cid
    fbase = wid * FPT
    pltpu.sync_copy(faces_hbm.at[pl.ds(fbase * 3, FPT * 3)], fwin)
    pltpu.sync_copy(vpl_hbm, vbuf)
    iot = lax.iota(jnp.int32, L)
    zero16 = jnp.zeros((L,), jnp.float32)
    one16 = jnp.ones((L,), jnp.float32)

    def zbody(i, carry):
        acc[pl.ds(i * L, L)] = zero16
        return carry

    lax.fori_loop(0, ACC // L, zbody, 0)

    def fill(g, krow, rrow, crow):
        fid, valid, rc = _expand_group(fwin, fbase, g, iot)
        for p, (r, c) in enumerate(rc):
            krow[pl.ds(p * L, L)] = jnp.where(valid, r * V + c, SENT)
            rrow[pl.ds(p * L, L)] = r
            crow[pl.ds(p * L, L)] = c

    def consume(g, krow, rrow, crow, trow):
        fid = fbase + g * L + iot
        for p in range(6):
            sl = pl.ds(p * L, L)
            k = krow[sl]
            r = rrow[sl]
            c = crow[sl]
            t = trow[sl]
            m = (t == fid * 6 + p) & (k < SENT)
            plsc.addupdate_scatter(acc, [r], one16, mask=m)
            for d in range(3):
                vg = plsc.load_gather(vbuf, [c + d * VPAD])
                plsc.addupdate_scatter(acc, [r + (d + 1) * VPAD], vg, mask=m)

    def body(i, carry):
        fill(2 * i, ka, ra, ca)
        da = pltpu.async_copy(t_hbm.at[ka], ta, sem)
        fill(2 * i + 1, kb, rb, cb)
        db = pltpu.async_copy(t_hbm.at[kb], tb, sem)
        da.wait()
        consume(2 * i, ka, ra, ca, ta)
        db.wait()
        consume(2 * i + 1, kb, rb, cb, tb)
        return carry

    lax.fori_loop(0, GROUPS // 2, body, 0)

    pltpu.sync_copy(acc, spm.at[sid])
    plsc.subcore_barrier()
    off = sid * SLICE
    pltpu.sync_copy(spm.at[0, pl.ds(off, SLICE)], sumb)
    for s in range(1, NS):
        pltpu.sync_copy(spm.at[s, pl.ds(off, SLICE)], tmpb)

        def abody(i, carry):
            sl = pl.ds(i * L, L)
            sumb[sl] = sumb[sl] + tmpb[sl]
            return carry

        lax.fori_loop(0, SLICE // L, abody, 0)
    pltpu.sync_copy(sumb, part_hbm.at[cid, pl.ds(off, SLICE)])


def _finalize_body(part_ref, vpl_ref, out_ref):
    p = part_ref[0] + part_ref[1]
    deg = p[0]
    s = None
    for d in range(3):
        lap = deg * vpl_ref[d] - p[d + 1]
        s = lap * lap if s is None else s + lap * lap
    out_ref[0, 0] = jnp.sum(s) * jnp.float32(WEIGHT / V)


_mesh = plsc.VectorSubcoreMesh(
    core_axis_name="c", subcore_axis_name="s", num_cores=NC, num_subcores=NS)

_sc_params = pltpu.CompilerParams(needs_layout_passes=False)

_stage1 = pl.kernel(
    _stage1_body,
    out_type=jax.ShapeDtypeStruct((TSIZE,), jnp.int32),
    mesh=_mesh,
    scratch_types=[
        pltpu.VMEM((FPT * 3,), jnp.int32),
        pltpu.VMEM((CHUNK,), jnp.int32), pltpu.VMEM((CHUNK,), jnp.int32),
        pltpu.VMEM((CHUNK,), jnp.int32), pltpu.VMEM((CHUNK,), jnp.int32),
        pltpu.SemaphoreType.DMA,
    ],
    compiler_params=_sc_params,
)

_stage2 = pl.kernel(
    _stage2_body,
    out_type=jax.ShapeDtypeStruct((NC, ACC), jnp.float32),
    mesh=_mesh,
    scratch_types=[
        pltpu.VMEM((FPT * 3,), jnp.int32),
        pltpu.VMEM((3 * VPAD,), jnp.float32),
        pltpu.VMEM((ACC,), jnp.float32),
        pltpu.VMEM((CHUNK,), jnp.int32), pltpu.VMEM((CHUNK,), jnp.int32),
        pltpu.VMEM((CHUNK,), jnp.int32), pltpu.VMEM((CHUNK,), jnp.int32),
        pltpu.VMEM((CHUNK,), jnp.int32), pltpu.VMEM((CHUNK,), jnp.int32),
        pltpu.VMEM((CHUNK,), jnp.int32), pltpu.VMEM((CHUNK,), jnp.int32),
        pltpu.VMEM_SHARED((NS, ACC), jnp.float32),
        pltpu.VMEM((SLICE,), jnp.float32),
        pltpu.VMEM((SLICE,), jnp.float32),
        pltpu.SemaphoreType.DMA,
    ],
    compiler_params=_sc_params,
)

_finalize = pl.pallas_call(
    _finalize_body,
    out_shape=jax.ShapeDtypeStruct((1, 1), jnp.float32),
    in_specs=[pl.BlockSpec(memory_space=pltpu.VMEM),
              pl.BlockSpec(memory_space=pltpu.VMEM)],
    out_specs=pl.BlockSpec(memory_space=pltpu.SMEM),
)


@jax.jit
def kernel(vertices, faces):
    faces_flat = jnp.pad(jnp.reshape(faces, (-1,)), (0, (FPAD - F) * 3))
    vpl = jnp.pad(jnp.transpose(vertices[0]), ((0, 0), (0, VPAD - V)))
    t = _stage1(faces_flat)
    part = _stage2(faces_flat, jnp.reshape(vpl, (-1,)), t)
    out = _finalize(jnp.reshape(part, (NC, 4, 80, 128)),
                    jnp.reshape(vpl, (3, 80, 128)))
    return out[0, 0]
